# baseline (speedup 1.0000x reference)
import jax
import jax.numpy as jnp
from jax.experimental import pallas as pl

_THRESHOLD = 0.5
_TM = 512


def _router_body(x_ref, wt_ref, b_ref, o_ref):
    z = jnp.dot(x_ref[...], wt_ref[...], preferred_element_type=jnp.float32)
    z = z + b_ref[...]
    s = jax.nn.sigmoid(z)
    o_ref[...] = jnp.where(s > _THRESHOLD, s, 0.0)


def kernel(x, W, b):
    Bb, S, D = x.shape
    H = W.shape[0]
    M = Bb * S
    xf = x.reshape(M, D)
    wt = W.T
    b2 = b.reshape(1, H)

    out = pl.pallas_call(
        _router_body,
        grid=(M // _TM,),
        in_specs=[
            pl.BlockSpec((_TM, D), lambda i: (i, 0)),
            pl.BlockSpec((D, H), lambda i: (0, 0)),
            pl.BlockSpec((1, H), lambda i: (0, 0)),
        ],
        out_specs=pl.BlockSpec((_TM, H), lambda i: (i, 0)),
        out_shape=jax.ShapeDtypeStruct((M, H), jnp.float32),
    )(xf, wt, b2)
    return out.reshape(Bb, S, H)
